# baseline (speedup 1.0000x reference)
import functools

import jax
import jax.numpy as jnp
from jax import lax
from jax.experimental import pallas as pl
from jax.experimental.pallas import tpu as pltpu
from jax.experimental.pallas import tpu_sc as plsc

N = 10000
E = 320000
D = 128
HID = 128

NC = 2
NS = 16
EB = 96
NPAD = 10240
EPAD = 322560
ROWS_PER_TILE = NPAD // NS
RB = 1000


def _lane_bcast(vec, lane):
  return lax.gather(
      vec, jnp.full((16, 1), lane, jnp.int32),
      lax.GatherDimensionNumbers(offset_dims=(),
                                 collapsed_slice_dims=(0,),
                                 start_index_map=(0,)),
      (1,), mode=lax.GatherScatterMode.PROMISE_IN_BOUNDS)


def _make_edge_pass(split_edges: bool):
  edges_per_core = EPAD // NC if split_edges else EPAD
  epb = edges_per_core // NS
  n_iter = epb // EB
  assert epb % EB == 0 and n_iter % 3 == 0
  NG = EB // 16

  mesh = plsc.VectorSubcoreMesh(core_axis_name="c", subcore_axis_name="s",
                                num_cores=NC, num_subcores=NS)

  @functools.partial(
      pl.kernel,
      out_type=(jax.ShapeDtypeStruct((NPAD, HID), jnp.float32),
                jax.ShapeDtypeStruct((NPAD, HID), jnp.float32)),
      mesh=mesh,
      scratch_types=[
          [pltpu.VMEM((EB,), jnp.int32)] * 3,
          [pltpu.VMEM((EB,), jnp.int32)] * 3,
          [pltpu.VMEM((EB,), jnp.float32)] * 3,
          [pltpu.VMEM((EB, HID), jnp.float32)] * 3,
          pltpu.VMEM_SHARED((NPAD, HID), jnp.float32),
          [pltpu.SemaphoreType.DMA] * 3,
          [pltpu.SemaphoreType.DMA] * 3,
          [pltpu.SemaphoreType.DMA] * 3,
      ],
  )
  def kern(p0_hbm, p1_hbm, src_hbm, dst_hbm, w_hbm, zrows_hbm,
           out0_hbm, out1_hbm, src_v, dst_v, w_v, rows_v, agg_s,
           msems, gsems, ssems):
    cid = lax.axis_index("c")
    sid = lax.axis_index("s")
    row0 = sid * ROWS_PER_TILE

    pltpu.sync_copy(zrows_hbm, agg_s.at[pl.ds(row0, ROWS_PER_TILE)])
    plsc.subcore_barrier()

    ebase = (cid * edges_per_core if split_edges else 0) + sid * epb

    def meta_copies(b, mslot):
      off = ebase + b * EB
      return (pltpu.make_async_copy(src_hbm.at[pl.ds(off, EB)], src_v[mslot],
                                    msems[mslot]),
              pltpu.make_async_copy(dst_hbm.at[pl.ds(off, EB)], dst_v[mslot],
                                    msems[mslot]),
              pltpu.make_async_copy(w_hbm.at[pl.ds(off, EB)], w_v[mslot],
                                    msems[mslot]))

    def meta_start(b, mslot):
      for d in meta_copies(b, mslot):
        d.start()

    def meta_wait(mslot):
      for d in meta_copies(0, mslot):
        d.wait()

    def process(p_hbm):
      def gather_copy(mslot, gslot):
        return pltpu.make_async_copy(p_hbm.at[src_v[mslot]], rows_v[gslot],
                                     gsems[gslot])

      def scatter_start(slot):
        pltpu.async_copy(rows_v[slot], agg_s.at[dst_v[slot]], ssems[slot],
                         add=True)

      def scatter_wait(slot):
        pltpu.make_async_copy(rows_v[slot], agg_s.at[dst_v[slot]],
                              ssems[slot]).wait()

      for u in range(3):
        meta_start(u, u)
      for u in range(2):
        meta_wait(u)
        gather_copy(u, u).start()

      def body(i, carry):
        for u in range(3):
          b = i * 3 + u
          gather_copy(u, u).wait()

          for g in range(NG):
            wv = w_v[u][pl.ds(g * 16, 16)]
            for e16 in range(16):
              wb = _lane_bcast(wv, e16)
              e = g * 16 + e16
              for j in range(HID // 16):
                rows_v[u][e, pl.ds(j * 16, 16)] = (
                    rows_v[u][e, pl.ds(j * 16, 16)] * wb)

          scatter_start(u)

          u2 = (u + 2) % 3
          pl.when(b >= 1)(lambda: scatter_wait(u2))
          meta_wait(u2)
          gather_copy(u2, u2).start()
          meta_start(lax.rem(b + 3, n_iter), u)
        return carry
      lax.fori_loop(0, n_iter // 3, body, 0)

      scatter_wait((n_iter - 1) % 3)
      for u in range(2):
        gather_copy(u, u).wait()
      meta_wait(2)

    pl.when(cid == 0)(lambda: process(p0_hbm))
    pl.when(cid == 1)(lambda: process(p1_hbm))
    plsc.subcore_barrier()

    def dump(out_hbm):
      pltpu.sync_copy(agg_s.at[pl.ds(row0, ROWS_PER_TILE)],
                      out_hbm.at[pl.ds(row0, ROWS_PER_TILE)])
    pl.when(cid == 0)(lambda: dump(out0_hbm))
    pl.when(cid == 1)(lambda: dump(out1_hbm))

  return kern


_edge_pass_zr = _make_edge_pass(split_edges=False)
_edge_pass_h = _make_edge_pass(split_edges=True)


def _row_spec(d):
  return pl.BlockSpec((RB, d), lambda i: (i, 0))


def _full_spec(shape):
  return pl.BlockSpec(shape, lambda i: (0,) * len(shape))


def _mm_zr(x, h, wx, wh):
  def body(x_ref, h_ref, wx_ref, wh_ref, sz_ref, pz_ref, sr_ref, pr_ref):
    s = (jnp.dot(x_ref[...], wx_ref[...], preferred_element_type=jnp.float32)
         + jnp.dot(h_ref[...], wh_ref[...], preferred_element_type=jnp.float32))
    sz_ref[...] = s[:, 0:128]
    pz_ref[...] = s[:, 128:256]
    sr_ref[...] = s[:, 256:384]
    pr_ref[...] = s[:, 384:512]

  return pl.pallas_call(
      body,
      grid=(N // RB,),
      in_specs=[_row_spec(D), _row_spec(HID),
                _full_spec((D, 4 * HID)), _full_spec((HID, 4 * HID))],
      out_specs=[_row_spec(HID)] * 4,
      out_shape=[jax.ShapeDtypeStruct((N, HID), jnp.float32)] * 4,
  )(x, h, wx, wh)


def _gates(sz, sr, aggz, aggr, deg2, x, h, wx, wh, bz2, br2):
  def body(sz_ref, sr_ref, az_ref, ar_ref, dg_ref, x_ref, h_ref,
           wx_ref, wh_ref, bz_ref, br_ref, z_ref, sh_ref, ph_ref):
    dinv = 1.0 / dg_ref[...]
    z = jax.nn.sigmoid(sz_ref[...] + az_ref[...] * dinv + bz_ref[...])
    r = jax.nn.sigmoid(sr_ref[...] + ar_ref[...] * dinv + br_ref[...])
    rh = r * h_ref[...]
    t = (jnp.dot(x_ref[...], wx_ref[...], preferred_element_type=jnp.float32)
         + jnp.dot(rh, wh_ref[...], preferred_element_type=jnp.float32))
    z_ref[...] = z
    sh_ref[...] = t[:, 0:128]
    ph_ref[...] = t[:, 128:256]

  return pl.pallas_call(
      body,
      grid=(N // RB,),
      in_specs=[_row_spec(HID), _row_spec(HID), _row_spec(HID), _row_spec(HID),
                _row_spec(1), _row_spec(D), _row_spec(HID),
                _full_spec((D, 2 * HID)), _full_spec((HID, 2 * HID)),
                _full_spec((1, HID)), _full_spec((1, HID))],
      out_specs=[_row_spec(HID)] * 3,
      out_shape=[jax.ShapeDtypeStruct((N, HID), jnp.float32)] * 3,
  )(sz, sr, aggz, aggr, deg2, x, h, wx, wh, bz2, br2)


def _final(z, sh, ah0, ah1, deg2, h, bh2, wlT, bl2):
  def body(z_ref, sh_ref, a0_ref, a1_ref, dg_ref, h_ref, bh_ref, wl_ref,
           bl_ref, out_ref):
    dinv = 1.0 / dg_ref[...]
    ht = jnp.tanh(sh_ref[...] + (a0_ref[...] + a1_ref[...]) * dinv
                  + bh_ref[...])
    z = z_ref[...]
    hnew = z * h_ref[...] + (1.0 - z) * ht
    hr = jnp.maximum(hnew, 0.0)
    out_ref[...] = (jnp.sum(hr * wl_ref[...], axis=1, keepdims=True)
                    + bl_ref[...])

  return pl.pallas_call(
      body,
      grid=(N // RB,),
      in_specs=[_row_spec(HID), _row_spec(HID), _row_spec(HID), _row_spec(HID),
                _row_spec(1), _row_spec(HID),
                _full_spec((1, HID)), _full_spec((1, HID)),
                _full_spec((1, 1))],
      out_specs=[_row_spec(1)],
      out_shape=[jax.ShapeDtypeStruct((N, 1), jnp.float32)],
  )(z, sh, ah0, ah1, deg2, h, bh2, wlT, bl2)[0]


def kernel(x, edge, edge_weight, prev_hidden_state, deg,
           Wz0, Wz1, bz, Wr0, Wr1, br, Wh0, Wh1, bh, Wl, bl):
  edge = edge.astype(jnp.int32)
  src, dst = edge[0], edge[1]
  h = prev_hidden_state
  deg2 = deg.reshape(N, 1)

  wzr_x = jnp.concatenate([Wz0[:D], Wz1[:D], Wr0[:D], Wr1[:D]], axis=1)
  wzr_h = jnp.concatenate([Wz0[D:], Wz1[D:], Wr0[D:], Wr1[D:]], axis=1)
  wh_x = jnp.concatenate([Wh0[:D], Wh1[:D]], axis=1)
  wh_h = jnp.concatenate([Wh0[D:], Wh1[D:]], axis=1)
  bz2 = bz.reshape(1, HID)
  br2 = br.reshape(1, HID)
  bh2 = bh.reshape(1, HID)
  wlT = Wl.reshape(1, HID)
  bl2 = bl.reshape(1, 1)
  zrows = jnp.zeros((ROWS_PER_TILE, HID), jnp.float32)

  npad_e = EPAD - E
  spread = (jnp.arange(npad_e, dtype=jnp.int32) * 97) % N
  srcp = jnp.concatenate([src, spread])
  dstp = jnp.concatenate([dst, spread])
  wp = jnp.concatenate([edge_weight, jnp.zeros((npad_e,), jnp.float32)])

  sz, pz, sr, pr = _mm_zr(x, h, wzr_x, wzr_h)
  aggz, aggr = _edge_pass_zr(pz, pr, srcp, dstp, wp, zrows)
  z, sh, ph = _gates(sz, sr, aggz, aggr, deg2, x, h, wh_x, wh_h, bz2, br2)
  ah0, ah1 = _edge_pass_h(ph, ph, srcp, dstp, wp, zrows)
  return _final(z, sh, ah0, ah1, deg2, h, bh2, wlT, bl2)


# speedup vs baseline: 1.1751x; 1.1751x over previous
import functools

import jax
import jax.numpy as jnp
from jax import lax
from jax.experimental import pallas as pl
from jax.experimental.pallas import tpu as pltpu
from jax.experimental.pallas import tpu_sc as plsc

N = 10000
E = 320000
D = 128
HID = 128

NC = 2
NS = 16
EB = 128
NPAD = 10240
EPAD = 327680
ROWS_PER_TILE = NPAD // NS
RB = 1000


def _lane_bcast(vec, lane):
  return lax.gather(
      vec, jnp.full((16, 1), lane, jnp.int32),
      lax.GatherDimensionNumbers(offset_dims=(),
                                 collapsed_slice_dims=(0,),
                                 start_index_map=(0,)),
      (1,), mode=lax.GatherScatterMode.PROMISE_IN_BOUNDS)


def _make_edge_pass(split_edges: bool):
  edges_per_core = EPAD // NC if split_edges else EPAD
  epb = edges_per_core // NS
  n_iter = epb // EB
  assert epb % EB == 0 and n_iter % 4 == 0
  NG = EB // 16

  mesh = plsc.VectorSubcoreMesh(core_axis_name="c", subcore_axis_name="s",
                                num_cores=NC, num_subcores=NS)

  @functools.partial(
      pl.kernel,
      out_type=(jax.ShapeDtypeStruct((NPAD, HID), jnp.float32),
                jax.ShapeDtypeStruct((NPAD, HID), jnp.float32)),
      mesh=mesh,
      scratch_types=[
          [pltpu.VMEM((EB,), jnp.int32)] * 4,
          [pltpu.VMEM((EB,), jnp.int32)] * 4,
          [pltpu.VMEM((EB,), jnp.float32)] * 4,
          [pltpu.VMEM((EB, HID), jnp.float32)] * 2,
          pltpu.VMEM_SHARED((NPAD, HID), jnp.float32),
          [pltpu.SemaphoreType.DMA] * 4,
          [pltpu.SemaphoreType.DMA] * 2,
      ],
  )
  def kern(p0_hbm, p1_hbm, src_hbm, dst_hbm, w_hbm, zrows_hbm,
           out0_hbm, out1_hbm, src_v, dst_v, w_v, rows_v, agg_s,
           msems, gsems):
    cid = lax.axis_index("c")
    sid = lax.axis_index("s")
    row0 = sid * ROWS_PER_TILE

    pltpu.sync_copy(zrows_hbm, agg_s.at[pl.ds(row0, ROWS_PER_TILE)])
    plsc.subcore_barrier()

    ebase = (cid * edges_per_core if split_edges else 0) + sid * epb

    def meta_copies(b, mslot):
      off = ebase + b * EB
      return (pltpu.make_async_copy(src_hbm.at[pl.ds(off, EB)], src_v[mslot],
                                    msems[mslot]),
              pltpu.make_async_copy(dst_hbm.at[pl.ds(off, EB)], dst_v[mslot],
                                    msems[mslot]),
              pltpu.make_async_copy(w_hbm.at[pl.ds(off, EB)], w_v[mslot],
                                    msems[mslot]))

    def meta_start(b, mslot):
      for d in meta_copies(b, mslot):
        d.start()

    def meta_wait(mslot):
      for d in meta_copies(0, mslot):
        d.wait()

    def process(p_hbm):
      def gather_copy(mslot, gslot):
        return pltpu.make_async_copy(p_hbm.at[src_v[mslot]], rows_v[gslot],
                                     gsems[gslot])

      for u in range(4):
        meta_start(u, u)
      for u in range(2):
        meta_wait(u)
        gather_copy(u, u).start()

      def body(i, carry):
        for u in range(4):
          b = i * 4 + u
          gslot = u % 2
          gather_copy(u, gslot).wait()

          def scale(g, c2):
            wv = w_v[u][pl.ds(g * 16, 16)]
            for e16 in range(16):
              wb = _lane_bcast(wv, e16)
              e = g * 16 + e16
              for j in range(HID // 16):
                rows_v[gslot][e, pl.ds(j * 16, 16)] = (
                    rows_v[gslot][e, pl.ds(j * 16, 16)] * wb)
            return c2
          lax.fori_loop(0, NG, scale, 0, unroll=2)

          pltpu.sync_copy(rows_v[gslot], agg_s.at[dst_v[u]], add=True)

          mslot2 = (u + 2) % 4
          meta_wait(mslot2)
          gather_copy(mslot2, gslot).start()
          meta_start(lax.rem(b + 4, n_iter), u)
        return carry
      lax.fori_loop(0, n_iter // 4, body, 0)

      for u in range(2):
        gather_copy(u, u).wait()
      for u in range(2, 4):
        meta_wait(u)

    pl.when(cid == 0)(lambda: process(p0_hbm))
    pl.when(cid == 1)(lambda: process(p1_hbm))
    plsc.subcore_barrier()

    def dump(out_hbm):
      pltpu.sync_copy(agg_s.at[pl.ds(row0, ROWS_PER_TILE)],
                      out_hbm.at[pl.ds(row0, ROWS_PER_TILE)])
    pl.when(cid == 0)(lambda: dump(out0_hbm))
    pl.when(cid == 1)(lambda: dump(out1_hbm))

  return kern


_edge_pass_zr = _make_edge_pass(split_edges=False)
_edge_pass_h = _make_edge_pass(split_edges=True)


def _row_spec(d):
  return pl.BlockSpec((RB, d), lambda i: (i, 0))


def _full_spec(shape):
  return pl.BlockSpec(shape, lambda i: (0,) * len(shape))


def _mm_zr(x, h, wx, wh):
  def body(x_ref, h_ref, wx_ref, wh_ref, sz_ref, pz_ref, sr_ref, pr_ref):
    s = (jnp.dot(x_ref[...], wx_ref[...], preferred_element_type=jnp.float32)
         + jnp.dot(h_ref[...], wh_ref[...], preferred_element_type=jnp.float32))
    sz_ref[...] = s[:, 0:128]
    pz_ref[...] = s[:, 128:256]
    sr_ref[...] = s[:, 256:384]
    pr_ref[...] = s[:, 384:512]

  return pl.pallas_call(
      body,
      grid=(N // RB,),
      in_specs=[_row_spec(D), _row_spec(HID),
                _full_spec((D, 4 * HID)), _full_spec((HID, 4 * HID))],
      out_specs=[_row_spec(HID)] * 4,
      out_shape=[jax.ShapeDtypeStruct((N, HID), jnp.float32)] * 4,
  )(x, h, wx, wh)


def _gates(sz, sr, aggz, aggr, deg2, x, h, wx, wh, bz2, br2):
  def body(sz_ref, sr_ref, az_ref, ar_ref, dg_ref, x_ref, h_ref,
           wx_ref, wh_ref, bz_ref, br_ref, z_ref, sh_ref, ph_ref):
    dinv = 1.0 / dg_ref[...]
    z = jax.nn.sigmoid(sz_ref[...] + az_ref[...] * dinv + bz_ref[...])
    r = jax.nn.sigmoid(sr_ref[...] + ar_ref[...] * dinv + br_ref[...])
    rh = r * h_ref[...]
    t = (jnp.dot(x_ref[...], wx_ref[...], preferred_element_type=jnp.float32)
         + jnp.dot(rh, wh_ref[...], preferred_element_type=jnp.float32))
    z_ref[...] = z
    sh_ref[...] = t[:, 0:128]
    ph_ref[...] = t[:, 128:256]

  return pl.pallas_call(
      body,
      grid=(N // RB,),
      in_specs=[_row_spec(HID), _row_spec(HID), _row_spec(HID), _row_spec(HID),
                _row_spec(1), _row_spec(D), _row_spec(HID),
                _full_spec((D, 2 * HID)), _full_spec((HID, 2 * HID)),
                _full_spec((1, HID)), _full_spec((1, HID))],
      out_specs=[_row_spec(HID)] * 3,
      out_shape=[jax.ShapeDtypeStruct((N, HID), jnp.float32)] * 3,
  )(sz, sr, aggz, aggr, deg2, x, h, wx, wh, bz2, br2)


def _final(z, sh, ah0, ah1, deg2, h, bh2, wlT, bl2):
  def body(z_ref, sh_ref, a0_ref, a1_ref, dg_ref, h_ref, bh_ref, wl_ref,
           bl_ref, out_ref):
    dinv = 1.0 / dg_ref[...]
    ht = jnp.tanh(sh_ref[...] + (a0_ref[...] + a1_ref[...]) * dinv
                  + bh_ref[...])
    z = z_ref[...]
    hnew = z * h_ref[...] + (1.0 - z) * ht
    hr = jnp.maximum(hnew, 0.0)
    out_ref[...] = (jnp.sum(hr * wl_ref[...], axis=1, keepdims=True)
                    + bl_ref[...])

  return pl.pallas_call(
      body,
      grid=(N // RB,),
      in_specs=[_row_spec(HID), _row_spec(HID), _row_spec(HID), _row_spec(HID),
                _row_spec(1), _row_spec(HID),
                _full_spec((1, HID)), _full_spec((1, HID)),
                _full_spec((1, 1))],
      out_specs=[_row_spec(1)],
      out_shape=[jax.ShapeDtypeStruct((N, 1), jnp.float32)],
  )(z, sh, ah0, ah1, deg2, h, bh2, wlT, bl2)[0]


def kernel(x, edge, edge_weight, prev_hidden_state, deg,
           Wz0, Wz1, bz, Wr0, Wr1, br, Wh0, Wh1, bh, Wl, bl):
  edge = edge.astype(jnp.int32)
  src, dst = edge[0], edge[1]
  h = prev_hidden_state
  deg2 = deg.reshape(N, 1)

  wzr_x = jnp.concatenate([Wz0[:D], Wz1[:D], Wr0[:D], Wr1[:D]], axis=1)
  wzr_h = jnp.concatenate([Wz0[D:], Wz1[D:], Wr0[D:], Wr1[D:]], axis=1)
  wh_x = jnp.concatenate([Wh0[:D], Wh1[:D]], axis=1)
  wh_h = jnp.concatenate([Wh0[D:], Wh1[D:]], axis=1)
  bz2 = bz.reshape(1, HID)
  br2 = br.reshape(1, HID)
  bh2 = bh.reshape(1, HID)
  wlT = Wl.reshape(1, HID)
  bl2 = bl.reshape(1, 1)
  zrows = jnp.zeros((ROWS_PER_TILE, HID), jnp.float32)

  npad_e = EPAD - E
  spread = (jnp.arange(npad_e, dtype=jnp.int32) * 97) % N
  srcp = jnp.concatenate([src, spread])
  dstp = jnp.concatenate([dst, spread])
  wp = jnp.concatenate([edge_weight, jnp.zeros((npad_e,), jnp.float32)])

  sz, pz, sr, pr = _mm_zr(x, h, wzr_x, wzr_h)
  aggz, aggr = _edge_pass_zr(pz, pr, srcp, dstp, wp, zrows)
  z, sh, ph = _gates(sz, sr, aggz, aggr, deg2, x, h, wh_x, wh_h, bz2, br2)
  ah0, ah1 = _edge_pass_h(ph, ph, srcp, dstp, wp, zrows)
  return _final(z, sh, ah0, ah1, deg2, h, bh2, wlT, bl2)


# speedup vs baseline: 1.1884x; 1.0113x over previous
import functools

import jax
import jax.numpy as jnp
from jax import lax
from jax.experimental import pallas as pl
from jax.experimental.pallas import tpu as pltpu
from jax.experimental.pallas import tpu_sc as plsc

N = 10000
E = 320000
D = 128
HID = 128

NC = 2
NS = 16
EB = 128
NPAD = 10240
EPAD = 327680
ROWS_PER_TILE = NPAD // NS
RB = 1000


def _lane_bcast(vec, lane):
  return lax.gather(
      vec, jnp.full((16, 1), lane, jnp.int32),
      lax.GatherDimensionNumbers(offset_dims=(),
                                 collapsed_slice_dims=(0,),
                                 start_index_map=(0,)),
      (1,), mode=lax.GatherScatterMode.PROMISE_IN_BOUNDS)


def _make_edge_pass(split_edges: bool):
  edges_per_core = EPAD // NC if split_edges else EPAD
  epb = edges_per_core // NS
  n_iter = epb // EB
  assert epb % EB == 0 and n_iter % 4 == 0
  NG = EB // 16

  mesh = plsc.VectorSubcoreMesh(core_axis_name="c", subcore_axis_name="s",
                                num_cores=NC, num_subcores=NS)

  @functools.partial(
      pl.kernel,
      out_type=(jax.ShapeDtypeStruct((NPAD, HID), jnp.float32),
                jax.ShapeDtypeStruct((NPAD, HID), jnp.float32)),
      mesh=mesh,
      scratch_types=[
          [pltpu.VMEM((EB,), jnp.int32)] * 4,
          [pltpu.VMEM((EB,), jnp.int32)] * 4,
          [pltpu.VMEM((EB,), jnp.float32)] * 4,
          [pltpu.VMEM((EB, HID), jnp.float32)] * 2,
          pltpu.VMEM_SHARED((NPAD, HID), jnp.float32),
          [pltpu.SemaphoreType.DMA] * 4,
          [pltpu.SemaphoreType.DMA] * 2,
      ],
  )
  def kern(p0_hbm, p1_hbm, src_hbm, dst_hbm, w_hbm, zrows_hbm,
           out0_hbm, out1_hbm, src_v, dst_v, w_v, rows_v, agg_s,
           msems, gsems):
    cid = lax.axis_index("c")
    sid = lax.axis_index("s")
    row0 = sid * ROWS_PER_TILE

    pltpu.sync_copy(zrows_hbm, agg_s.at[pl.ds(row0, ROWS_PER_TILE)])
    plsc.subcore_barrier()

    ebase = (cid * edges_per_core if split_edges else 0) + sid * epb

    def meta_copies(b, mslot):
      off = ebase + b * EB
      return (pltpu.make_async_copy(src_hbm.at[pl.ds(off, EB)], src_v[mslot],
                                    msems[mslot]),
              pltpu.make_async_copy(dst_hbm.at[pl.ds(off, EB)], dst_v[mslot],
                                    msems[mslot]),
              pltpu.make_async_copy(w_hbm.at[pl.ds(off, EB)], w_v[mslot],
                                    msems[mslot]))

    def meta_start(b, mslot):
      for d in meta_copies(b, mslot):
        d.start()

    def meta_wait(mslot):
      for d in meta_copies(0, mslot):
        d.wait()

    def process(p_hbm):
      def gather_copy(mslot, gslot):
        return pltpu.make_async_copy(p_hbm.at[src_v[mslot]], rows_v[gslot],
                                     gsems[gslot])

      for u in range(4):
        meta_start(u, u)
      for u in range(2):
        meta_wait(u)
        gather_copy(u, u).start()

      def body(i, carry):
        for u in range(4):
          b = i * 4 + u
          gslot = u % 2
          gather_copy(u, gslot).wait()

          def scale(g, c2):
            wv = w_v[u][pl.ds(g * 16, 16)]
            for e16 in range(16):
              wb = _lane_bcast(wv, e16)
              e = g * 16 + e16
              for j in range(HID // 16):
                rows_v[gslot][e, pl.ds(j * 16, 16)] = (
                    rows_v[gslot][e, pl.ds(j * 16, 16)] * wb)
            return c2
          lax.fori_loop(0, NG, scale, 0)

          pltpu.sync_copy(rows_v[gslot], agg_s.at[dst_v[u]], add=True)

          mslot2 = (u + 2) % 4
          meta_wait(mslot2)
          gather_copy(mslot2, gslot).start()
          meta_start(lax.rem(b + 4, n_iter), u)
        return carry
      lax.fori_loop(0, n_iter // 4, body, 0)

      for u in range(2):
        gather_copy(u, u).wait()
      for u in range(2, 4):
        meta_wait(u)

    pl.when(cid == 0)(lambda: process(p0_hbm))
    pl.when(cid == 1)(lambda: process(p1_hbm))
    plsc.subcore_barrier()

    def dump(out_hbm):
      pltpu.sync_copy(agg_s.at[pl.ds(row0, ROWS_PER_TILE)],
                      out_hbm.at[pl.ds(row0, ROWS_PER_TILE)])
    pl.when(cid == 0)(lambda: dump(out0_hbm))
    pl.when(cid == 1)(lambda: dump(out1_hbm))

  return kern


_edge_pass_zr = _make_edge_pass(split_edges=False)
_edge_pass_h = _make_edge_pass(split_edges=True)


def _row_spec(d):
  return pl.BlockSpec((RB, d), lambda i: (i, 0))


def _full_spec(shape):
  return pl.BlockSpec(shape, lambda i: (0,) * len(shape))


def _mm_zr(x, h, wx, wh):
  def body(x_ref, h_ref, wx_ref, wh_ref, sz_ref, pz_ref, sr_ref, pr_ref):
    s = (jnp.dot(x_ref[...], wx_ref[...], preferred_element_type=jnp.float32)
         + jnp.dot(h_ref[...], wh_ref[...], preferred_element_type=jnp.float32))
    sz_ref[...] = s[:, 0:128]
    pz_ref[...] = s[:, 128:256]
    sr_ref[...] = s[:, 256:384]
    pr_ref[...] = s[:, 384:512]

  return pl.pallas_call(
      body,
      grid=(N // RB,),
      in_specs=[_row_spec(D), _row_spec(HID),
                _full_spec((D, 4 * HID)), _full_spec((HID, 4 * HID))],
      out_specs=[_row_spec(HID)] * 4,
      out_shape=[jax.ShapeDtypeStruct((N, HID), jnp.float32)] * 4,
  )(x, h, wx, wh)


def _gates(sz, sr, aggz, aggr, deg2, x, h, wx, wh, bz2, br2):
  def body(sz_ref, sr_ref, az_ref, ar_ref, dg_ref, x_ref, h_ref,
           wx_ref, wh_ref, bz_ref, br_ref, z_ref, sh_ref, ph_ref):
    dinv = 1.0 / dg_ref[...]
    z = jax.nn.sigmoid(sz_ref[...] + az_ref[...] * dinv + bz_ref[...])
    r = jax.nn.sigmoid(sr_ref[...] + ar_ref[...] * dinv + br_ref[...])
    rh = r * h_ref[...]
    t = (jnp.dot(x_ref[...], wx_ref[...], preferred_element_type=jnp.float32)
         + jnp.dot(rh, wh_ref[...], preferred_element_type=jnp.float32))
    z_ref[...] = z
    sh_ref[...] = t[:, 0:128]
    ph_ref[...] = t[:, 128:256]

  return pl.pallas_call(
      body,
      grid=(N // RB,),
      in_specs=[_row_spec(HID), _row_spec(HID), _row_spec(HID), _row_spec(HID),
                _row_spec(1), _row_spec(D), _row_spec(HID),
                _full_spec((D, 2 * HID)), _full_spec((HID, 2 * HID)),
                _full_spec((1, HID)), _full_spec((1, HID))],
      out_specs=[_row_spec(HID)] * 3,
      out_shape=[jax.ShapeDtypeStruct((N, HID), jnp.float32)] * 3,
  )(sz, sr, aggz, aggr, deg2, x, h, wx, wh, bz2, br2)


def _final(z, sh, ah0, ah1, deg2, h, bh2, wlT, bl2):
  def body(z_ref, sh_ref, a0_ref, a1_ref, dg_ref, h_ref, bh_ref, wl_ref,
           bl_ref, out_ref):
    dinv = 1.0 / dg_ref[...]
    ht = jnp.tanh(sh_ref[...] + (a0_ref[...] + a1_ref[...]) * dinv
                  + bh_ref[...])
    z = z_ref[...]
    hnew = z * h_ref[...] + (1.0 - z) * ht
    hr = jnp.maximum(hnew, 0.0)
    out_ref[...] = (jnp.sum(hr * wl_ref[...], axis=1, keepdims=True)
                    + bl_ref[...])

  return pl.pallas_call(
      body,
      grid=(N // RB,),
      in_specs=[_row_spec(HID), _row_spec(HID), _row_spec(HID), _row_spec(HID),
                _row_spec(1), _row_spec(HID),
                _full_spec((1, HID)), _full_spec((1, HID)),
                _full_spec((1, 1))],
      out_specs=[_row_spec(1)],
      out_shape=[jax.ShapeDtypeStruct((N, 1), jnp.float32)],
  )(z, sh, ah0, ah1, deg2, h, bh2, wlT, bl2)[0]


def kernel(x, edge, edge_weight, prev_hidden_state, deg,
           Wz0, Wz1, bz, Wr0, Wr1, br, Wh0, Wh1, bh, Wl, bl):
  edge = edge.astype(jnp.int32)
  src, dst = edge[0], edge[1]
  h = prev_hidden_state
  deg2 = deg.reshape(N, 1)

  wzr_x = jnp.concatenate([Wz0[:D], Wz1[:D], Wr0[:D], Wr1[:D]], axis=1)
  wzr_h = jnp.concatenate([Wz0[D:], Wz1[D:], Wr0[D:], Wr1[D:]], axis=1)
  wh_x = jnp.concatenate([Wh0[:D], Wh1[:D]], axis=1)
  wh_h = jnp.concatenate([Wh0[D:], Wh1[D:]], axis=1)
  bz2 = bz.reshape(1, HID)
  br2 = br.reshape(1, HID)
  bh2 = bh.reshape(1, HID)
  wlT = Wl.reshape(1, HID)
  bl2 = bl.reshape(1, 1)
  zrows = jnp.zeros((ROWS_PER_TILE, HID), jnp.float32)

  npad_e = EPAD - E
  spread = (jnp.arange(npad_e, dtype=jnp.int32) * 97) % N
  srcp = jnp.concatenate([src, spread])
  dstp = jnp.concatenate([dst, spread])
  wp = jnp.concatenate([edge_weight, jnp.zeros((npad_e,), jnp.float32)])

  sz, pz, sr, pr = _mm_zr(x, h, wzr_x, wzr_h)
  aggz, aggr = _edge_pass_zr(pz, pr, srcp, dstp, wp, zrows)
  z, sh, ph = _gates(sz, sr, aggz, aggr, deg2, x, h, wh_x, wh_h, bz2, br2)
  ah0, ah1 = _edge_pass_h(ph, ph, srcp, dstp, wp, zrows)
  return _final(z, sh, ah0, ah1, deg2, h, bh2, wlT, bl2)
